# baseline (speedup 1.0000x reference)
import functools

import jax
import jax.numpy as jnp
from jax.experimental import pallas as pl
from jax.experimental.pallas import tpu as pltpu

_E = 8
_TOP_K = 2

_CB = 512
_FB = 1024


def _ffn_body(x_ref, s_ref, w1_ref, w2_ref, o_ref, acc_ref):
    f = pl.program_id(2)
    nf = pl.num_programs(2)

    x = x_ref[0] * s_ref[0, 0][:, None]
    xb = x.astype(jnp.bfloat16)
    h = jnp.dot(xb, w1_ref[0], preferred_element_type=jnp.float32)
    h = h * jax.nn.sigmoid(h)
    part = jnp.dot(h.astype(jnp.bfloat16), w2_ref[0],
                   preferred_element_type=jnp.float32)

    @pl.when(f == 0)
    def _():
        acc_ref[...] = part

    @pl.when(f != 0)
    def _():
        acc_ref[...] += part

    @pl.when(f == nf - 1)
    def _():
        o_ref[0] = acc_ref[...]


def _ffn(routed, scores, w1b, w2b, *, interpret=False):
    e, c, d = routed.shape
    dff = w1b.shape[2]
    grid = (e, c // _CB, dff // _FB)
    return pl.pallas_call(
        _ffn_body,
        grid=grid,
        in_specs=[
            pl.BlockSpec((1, _CB, d), lambda e, i, f: (e, i, 0)),
            pl.BlockSpec((1, 1, _CB), lambda e, i, f: (e, 0, i)),
            pl.BlockSpec((1, d, _FB), lambda e, i, f: (e, 0, f)),
            pl.BlockSpec((1, _FB, d), lambda e, i, f: (e, f, 0)),
        ],
        out_specs=pl.BlockSpec((1, _CB, d), lambda e, i, f: (e, i, 0)),
        out_shape=jax.ShapeDtypeStruct((e, c, d), jnp.float32),
        scratch_shapes=[pltpu.VMEM((_CB, d), jnp.float32)],
        compiler_params=pltpu.CompilerParams(
            dimension_semantics=("arbitrary", "arbitrary", "arbitrary"),
        ),
        interpret=interpret,
    )(routed, scores.reshape(e, 1, c), w1b, w2b)


def kernel(x, w_router, w1, w2):
    bz, slen, dim = x.shape
    xf = x.reshape(bz * slen, dim)
    n_tokens = xf.shape[0]
    capacity = (n_tokens * _TOP_K) // _E

    logits = xf @ w_router
    scores = jax.nn.softmax(logits, axis=-1)
    top_scores, selected = jax.lax.top_k(scores.T, capacity)
    token_indices = selected.reshape(-1)

    routed = jnp.take(xf, token_indices, axis=0).reshape(_E, capacity, dim)
    routed_out = _ffn(routed, top_scores,
                      w1.astype(jnp.bfloat16), w2.astype(jnp.bfloat16))

    out = jnp.zeros_like(xf)
    out = out.at[token_indices].add(routed_out.reshape(-1, dim))
    return out.reshape(bz, slen, dim)
